# Initial kernel scaffold; baseline (speedup 1.0000x reference)
import jax
import jax.numpy as jnp
from jax.experimental import pallas as pl


def kernel(node_feat, rel_emb, edge_index, edge_type, W_self, W_forward, W_rel, bias):
    raise NotImplementedError("write your pallas kernel here")


# speedup vs baseline: 6.5919x; 6.5919x over previous
import functools

import jax
import jax.numpy as jnp
from jax import lax
from jax.experimental import pallas as pl
from jax.experimental.pallas import tpu as pltpu
from jax.experimental.pallas import tpu_sc as plsc

NC = 2
NS = 16
NW = NC * NS
CH = 128


def _sc_agg_body(EPW, ROWS_PT, node_hbm, rel_hbm, src_hbm, dst_hbm, et_hbm,
                 acc_hbm, idx_s, idx_d, idx_e, h_v, r_v, acc_sh, sem1, sem2):
    cid = lax.axis_index("c")
    sid = lax.axis_index("s")
    wid = cid * NS + sid

    zeros16 = jnp.zeros((16,), jnp.float32)

    def zrow(r, c):
        for j in range(8):
            h_v[r, pl.ds(j * 16, 16)] = zeros16
        return c

    lax.fori_loop(0, CH, zrow, 0)

    def zcopy(k, c):
        pltpu.sync_copy(h_v, acc_sh.at[pl.ds(sid * ROWS_PT + k * CH, CH)])
        return c

    lax.fori_loop(0, ROWS_PT // CH, zcopy, 0)
    plsc.subcore_barrier()

    def chunk(i, c):
        base = wid * EPW + i * CH
        pltpu.sync_copy(src_hbm.at[pl.ds(base, CH)], idx_s)
        pltpu.sync_copy(dst_hbm.at[pl.ds(base, CH)], idx_d)
        pltpu.sync_copy(et_hbm.at[pl.ds(base, CH)], idx_e)
        cp1 = pltpu.async_copy(node_hbm.at[idx_s], h_v, sem1)
        cp2 = pltpu.async_copy(rel_hbm.at[idx_e], r_v, sem2)
        cp1.wait()
        cp2.wait()

        def sub_row(r, cc):
            for j in range(8):
                sl = pl.ds(j * 16, 16)
                h_v[r, sl] = h_v[r, sl] - r_v[r, sl]
            return cc

        lax.fori_loop(0, CH, sub_row, 0)
        pltpu.sync_copy(h_v, acc_sh.at[idx_d], add=True)
        return c

    lax.fori_loop(0, EPW // CH, chunk, 0)
    plsc.subcore_barrier()

    pltpu.sync_copy(acc_sh.at[pl.ds(sid * ROWS_PT, ROWS_PT)],
                    acc_hbm.at[cid, pl.ds(sid * ROWS_PT, ROWS_PT)])


def _tc_body(x_ref, a0_ref, a1_ref, rel_ref, ws_ref, wf_ref, wr_ref, b_ref,
             o_ref, rout_ref):
    a = a0_ref[0] + a1_ref[0]
    acc = jnp.dot(x_ref[...], ws_ref[...], preferred_element_type=jnp.float32)
    acc = acc + jnp.dot(a, wf_ref[...], preferred_element_type=jnp.float32)
    o_ref[...] = acc + b_ref[...]

    @pl.when(pl.program_id(0) == 0)
    def _():
        rout_ref[...] = jnp.dot(rel_ref[...], wr_ref[...],
                                preferred_element_type=jnp.float32)


def kernel(node_feat, rel_emb, edge_index, edge_type, W_self, W_forward,
           W_rel, bias):
    N, D = node_feat.shape
    R = rel_emb.shape[0]
    OUT = W_self.shape[0]
    E = edge_index.shape[1]
    EF = E // 2

    EPW = -(-EF // (NW * CH)) * CH
    NP = -(-(N + 1) // (NS * CH)) * (NS * CH)
    ROWS_PT = NP // NS
    pad = NW * EPW - EF

    src = jnp.concatenate([edge_index[0, :EF],
                           jnp.zeros((pad,), jnp.int32)])
    dst = jnp.concatenate([edge_index[1, :EF],
                           jnp.full((pad,), N, jnp.int32)])
    et = jnp.concatenate([edge_type[:EF], jnp.zeros((pad,), jnp.int32)])

    mesh = plsc.VectorSubcoreMesh(core_axis_name="c", subcore_axis_name="s",
                                  num_cores=NC, num_subcores=NS)
    sc_agg = pl.kernel(
        functools.partial(_sc_agg_body, EPW, ROWS_PT),
        out_type=jax.ShapeDtypeStruct((NC, NP, D), jnp.float32),
        mesh=mesh,
        scratch_types=[
            pltpu.VMEM((CH,), jnp.int32),
            pltpu.VMEM((CH,), jnp.int32),
            pltpu.VMEM((CH,), jnp.int32),
            pltpu.VMEM((CH, D), jnp.float32),
            pltpu.VMEM((CH, D), jnp.float32),
            pltpu.VMEM_SHARED((NP, D), jnp.float32),
            pltpu.SemaphoreType.DMA,
            pltpu.SemaphoreType.DMA,
        ],
    )
    acc = sc_agg(node_feat, rel_emb, src, dst, et)

    BN = 1000
    grid = N // BN
    out, rel_out = pl.pallas_call(
        _tc_body,
        grid=(grid,),
        in_specs=[
            pl.BlockSpec((BN, D), lambda i: (i, 0)),
            pl.BlockSpec((1, BN, D), lambda i: (0, i, 0)),
            pl.BlockSpec((1, BN, D), lambda i: (1, i, 0)),
            pl.BlockSpec((R, D), lambda i: (0, 0)),
            pl.BlockSpec((D, OUT), lambda i: (0, 0)),
            pl.BlockSpec((D, OUT), lambda i: (0, 0)),
            pl.BlockSpec((D, OUT), lambda i: (0, 0)),
            pl.BlockSpec((1, OUT), lambda i: (0, 0)),
        ],
        out_specs=[
            pl.BlockSpec((BN, OUT), lambda i: (i, 0)),
            pl.BlockSpec((R, OUT), lambda i: (0, 0)),
        ],
        out_shape=[
            jax.ShapeDtypeStruct((N, OUT), jnp.float32),
            jax.ShapeDtypeStruct((R, OUT), jnp.float32),
        ],
    )(node_feat, acc, acc, rel_emb, W_self.T, W_forward.T, W_rel.T,
      bias.reshape(1, OUT))
    return (out, rel_out)


# speedup vs baseline: 8.2252x; 1.2478x over previous
import functools

import jax
import jax.numpy as jnp
from jax import lax
from jax.experimental import pallas as pl
from jax.experimental.pallas import tpu as pltpu
from jax.experimental.pallas import tpu_sc as plsc

NC = 2
NS = 16
NW = NC * NS
CH = 64


def _sc_agg_body(EPW, ROWS_PT, node_hbm, rel_hbm, src_hbm, dst_hbm, et_hbm,
                 acc_hbm, idx_sA, idx_dA, idx_eA, h_vA, r_vA,
                 idx_sB, idx_dB, idx_eB, h_vB, r_vB,
                 acc_sh, semA, semB):
    cid = lax.axis_index("c")
    sid = lax.axis_index("s")
    wid = cid * NS + sid
    NCHUNK = EPW // CH

    zeros16 = jnp.zeros((16,), jnp.float32)

    def zrow(r, c):
        for j in range(8):
            h_vA[r, pl.ds(j * 16, 16)] = zeros16
        return c

    lax.fori_loop(0, CH, zrow, 0)

    def zcopy(k, c):
        pltpu.sync_copy(h_vA, acc_sh.at[pl.ds(sid * ROWS_PT + k * CH, CH)])
        return c

    lax.fori_loop(0, ROWS_PT // CH, zcopy, 0)
    plsc.subcore_barrier()

    ebase = wid * EPW

    def stage(chunk_i, idx_s, idx_d, idx_e, h_v, r_v, sem):
        base = ebase + chunk_i * CH
        pltpu.sync_copy(src_hbm.at[pl.ds(base, CH)], idx_s)
        pltpu.sync_copy(dst_hbm.at[pl.ds(base, CH)], idx_d)
        pltpu.sync_copy(et_hbm.at[pl.ds(base, CH)], idx_e)
        pltpu.async_copy(node_hbm.at[idx_s], h_v, sem)
        pltpu.async_copy(rel_hbm.at[idx_e], r_v, sem)

    def drain(h_v, r_v, sem):
        pltpu.make_async_copy(node_hbm.at[idx_sA], h_v, sem).wait()
        pltpu.make_async_copy(rel_hbm.at[idx_eA], r_v, sem).wait()

    def compute(idx_d, h_v, r_v):
        def sub_row(r, cc):
            for j in range(8):
                sl = pl.ds(j * 16, 16)
                h_v[r, sl] = h_v[r, sl] - r_v[r, sl]
            return cc

        lax.fori_loop(0, CH, sub_row, 0)
        pltpu.sync_copy(h_v, acc_sh.at[idx_d], add=True)

    stage(0, idx_sA, idx_dA, idx_eA, h_vA, r_vA, semA)

    def pipe(j, c):
        c0 = 2 * j
        stage(c0 + 1, idx_sB, idx_dB, idx_eB, h_vB, r_vB, semB)
        drain(h_vA, r_vA, semA)
        compute(idx_dA, h_vA, r_vA)

        @pl.when(c0 + 2 < NCHUNK)
        def _():
            stage(c0 + 2, idx_sA, idx_dA, idx_eA, h_vA, r_vA, semA)

        drain(h_vB, r_vB, semB)
        compute(idx_dB, h_vB, r_vB)
        return c

    lax.fori_loop(0, NCHUNK // 2, pipe, 0)
    plsc.subcore_barrier()

    pltpu.sync_copy(acc_sh.at[pl.ds(sid * ROWS_PT, ROWS_PT)],
                    acc_hbm.at[cid, pl.ds(sid * ROWS_PT, ROWS_PT)])


def _tc_body(x_ref, a0_ref, a1_ref, rel_ref, ws_ref, wf_ref, wr_ref, b_ref,
             o_ref, rout_ref):
    a = a0_ref[0] + a1_ref[0]
    acc = jnp.dot(x_ref[...], ws_ref[...], preferred_element_type=jnp.float32)
    acc = acc + jnp.dot(a, wf_ref[...], preferred_element_type=jnp.float32)
    o_ref[...] = acc + b_ref[...]

    @pl.when(pl.program_id(0) == 0)
    def _():
        rout_ref[...] = jnp.dot(rel_ref[...], wr_ref[...],
                                preferred_element_type=jnp.float32)


def kernel(node_feat, rel_emb, edge_index, edge_type, W_self, W_forward,
           W_rel, bias):
    N, D = node_feat.shape
    R = rel_emb.shape[0]
    OUT = W_self.shape[0]
    E = edge_index.shape[1]
    EF = E // 2

    EPW = -(-EF // (NW * 2 * CH)) * (2 * CH)
    NP = -(-(N + 1) // (NS * CH)) * (NS * CH)
    ROWS_PT = NP // NS
    pad = NW * EPW - EF

    src = jnp.concatenate([edge_index[0, :EF],
                           jnp.zeros((pad,), jnp.int32)])
    dummy_dst = N + jnp.arange(pad, dtype=jnp.int32) % (NP - N)
    dst = jnp.concatenate([edge_index[1, :EF], dummy_dst])
    et = jnp.concatenate([edge_type[:EF], jnp.zeros((pad,), jnp.int32)])

    mesh = plsc.VectorSubcoreMesh(core_axis_name="c", subcore_axis_name="s",
                                  num_cores=NC, num_subcores=NS)
    sc_agg = pl.kernel(
        functools.partial(_sc_agg_body, EPW, ROWS_PT),
        out_type=jax.ShapeDtypeStruct((NC, NP, D), jnp.float32),
        mesh=mesh,
        scratch_types=[
            pltpu.VMEM((CH,), jnp.int32),
            pltpu.VMEM((CH,), jnp.int32),
            pltpu.VMEM((CH,), jnp.int32),
            pltpu.VMEM((CH, D), jnp.float32),
            pltpu.VMEM((CH, D), jnp.float32),
            pltpu.VMEM((CH,), jnp.int32),
            pltpu.VMEM((CH,), jnp.int32),
            pltpu.VMEM((CH,), jnp.int32),
            pltpu.VMEM((CH, D), jnp.float32),
            pltpu.VMEM((CH, D), jnp.float32),
            pltpu.VMEM_SHARED((NP, D), jnp.float32),
            pltpu.SemaphoreType.DMA,
            pltpu.SemaphoreType.DMA,
        ],
    )
    acc = sc_agg(node_feat, rel_emb, src, dst, et)

    BN = 1000
    grid = N // BN
    out, rel_out = pl.pallas_call(
        _tc_body,
        grid=(grid,),
        in_specs=[
            pl.BlockSpec((BN, D), lambda i: (i, 0)),
            pl.BlockSpec((1, BN, D), lambda i: (0, i, 0)),
            pl.BlockSpec((1, BN, D), lambda i: (1, i, 0)),
            pl.BlockSpec((R, D), lambda i: (0, 0)),
            pl.BlockSpec((D, OUT), lambda i: (0, 0)),
            pl.BlockSpec((D, OUT), lambda i: (0, 0)),
            pl.BlockSpec((D, OUT), lambda i: (0, 0)),
            pl.BlockSpec((1, OUT), lambda i: (0, 0)),
        ],
        out_specs=[
            pl.BlockSpec((BN, OUT), lambda i: (i, 0)),
            pl.BlockSpec((R, OUT), lambda i: (0, 0)),
        ],
        out_shape=[
            jax.ShapeDtypeStruct((N, OUT), jnp.float32),
            jax.ShapeDtypeStruct((R, OUT), jnp.float32),
        ],
    )(node_feat, acc, acc, rel_emb, W_self.T, W_forward.T, W_rel.T,
      bias.reshape(1, OUT))
    return (out, rel_out)
